# baseline (speedup 1.0000x reference)
import functools

import jax
import jax.numpy as jnp
from jax import lax
from jax.experimental import pallas as pl
from jax.experimental.pallas import tpu as pltpu
from jax.experimental.pallas import tpu_sc as plsc

NC, NS = 2, 16
NW = NC * NS
D = 128
CPW = D // NW
KI = 2048
BLK = 2048
L = 16


def _round_up(a, b):
    return (a + b - 1) // b * b


def _stage_a(xt_ref, w1t_ref, o_ref):
    o_ref[...] = jnp.dot(w1t_ref[...], xt_ref[...],
                         preferred_element_type=jnp.float32)


def _stage_b(eacct, dege, yt, xwt, xt, w2t, etop_o, ewt_o, ewselft_o):
    rec = 1.0 / jnp.maximum(dege[...][0], 1.0)
    etopt = jnp.maximum(eacct[...] * rec + yt[...], 0.0)
    etop_o[...] = etopt.T
    ewt_o[...] = jnp.dot(w2t[...], etopt, preferred_element_type=jnp.float32)
    eselft = jnp.maximum(xwt[...] + xt[...], 0.0)
    ewselft_o[...] = jnp.dot(w2t[...], eselft,
                             preferred_element_type=jnp.float32)


def _stage_c(nacct, ewselft, degv, n_o):
    rec = 1.0 / (degv[...][0] + 1.0)
    n_o[...] = jnp.maximum((nacct[...] + ewselft[...]) * rec, 0.0).T


def _sc_pass(np_, nnzp, with_deg):
    npairs = nnzp // (2 * KI)
    ngrp = KI // L

    def body(*refs):
        if with_deg:
            (tabt, gidx, sidx, acct_out, dege_out, degv_out,
             t0, t1, t2, t3, a0, a1, a2, a3,
             gb0, gb1, sb0, sb1, degeb, degvb, semg, sems) = refs
        else:
            (tabt, gidx, sidx, acct_out,
             t0, t1, t2, t3, a0, a1, a2, a3,
             gb0, gb1, sb0, sb1, semg, sems) = refs
        tabs = (t0, t1, t2, t3)
        accs = (a0, a1, a2, a3)
        c = lax.axis_index("c")
        s = lax.axis_index("s")
        wid = s * NC + c
        w4 = wid * CPW

        for j in range(CPW):
            pltpu.sync_copy(tabt.at[w4 + j], tabs[j])

        zero16 = jnp.zeros((L,), jnp.float32)

        def zbody(i, carry):
            for a in accs:
                a[pl.ds(i * L, L)] = zero16
            if with_deg:
                degeb[pl.ds(i * L, L)] = zero16
                degvb[pl.ds(i * L, L)] = zero16
            return carry

        lax.fori_loop(0, np_ // L, zbody, 0)

        ones16 = jnp.ones((L,), jnp.float32)

        def process(gb, sb):
            def grp(g, carry):
                gi = gb[pl.ds(g * L, L)]
                si = sb[pl.ds(g * L, L)]
                for j in range(CPW):
                    v = plsc.load_gather(tabs[j], [gi])
                    plsc.addupdate_scatter(accs[j], [si], v)
                if with_deg:
                    plsc.addupdate_scatter(degeb, [si], ones16)
                    plsc.addupdate_scatter(degvb, [gi], ones16)
                return carry

            lax.fori_loop(0, ngrp, grp, 0)

        pltpu.async_copy(gidx.at[pl.ds(0, KI)], gb0, semg)
        pltpu.async_copy(sidx.at[pl.ds(0, KI)], sb0, sems)

        def pair(p, carry):
            t0off = pl.multiple_of(2 * p * KI, 8)
            t1off = pl.multiple_of((2 * p + 1) * KI, 8)
            t2off = pl.multiple_of((2 * p + 2) * KI, 8)
            pltpu.make_async_copy(gidx.at[pl.ds(t0off, KI)], gb0, semg).wait()
            pltpu.make_async_copy(sidx.at[pl.ds(t0off, KI)], sb0, sems).wait()
            pltpu.async_copy(gidx.at[pl.ds(t1off, KI)], gb1, semg)
            pltpu.async_copy(sidx.at[pl.ds(t1off, KI)], sb1, sems)
            process(gb0, sb0)
            pltpu.make_async_copy(gidx.at[pl.ds(t1off, KI)], gb1, semg).wait()
            pltpu.make_async_copy(sidx.at[pl.ds(t1off, KI)], sb1, sems).wait()

            @pl.when(p + 1 < npairs)
            def _():
                pltpu.async_copy(gidx.at[pl.ds(t2off, KI)], gb0, semg)
                pltpu.async_copy(sidx.at[pl.ds(t2off, KI)], sb0, sems)

            process(gb1, sb1)
            return carry

        lax.fori_loop(0, npairs, pair, 0)

        for j in range(CPW):
            pltpu.sync_copy(accs[j], acct_out.at[w4 + j])
        if with_deg:
            @pl.when(wid == 0)
            def _():
                pltpu.sync_copy(degeb, dege_out)
                pltpu.sync_copy(degvb, degv_out)

    if with_deg:
        out_type = (
            jax.ShapeDtypeStruct((D, np_), jnp.float32),
            jax.ShapeDtypeStruct((np_,), jnp.float32),
            jax.ShapeDtypeStruct((np_,), jnp.float32),
        )
    else:
        out_type = jax.ShapeDtypeStruct((D, np_), jnp.float32)

    scratch = (
        [pltpu.VMEM((np_,), jnp.float32) for _ in range(2 * CPW)]
        + [pltpu.VMEM((KI,), jnp.int32) for _ in range(4)]
        + ([pltpu.VMEM((np_,), jnp.float32) for _ in range(2)]
           if with_deg else [])
        + [pltpu.SemaphoreType.DMA, pltpu.SemaphoreType.DMA]
    )

    return pl.kernel(
        body,
        out_type=out_type,
        mesh=plsc.VectorSubcoreMesh(core_axis_name="c", subcore_axis_name="s"),
        scratch_types=scratch,
        compiler_params=pltpu.CompilerParams(needs_layout_passes=False),
    )


def kernel(x, y, hyperedge_index, W1, W2):
    num_nodes = x.shape[0]
    num_edges = y.shape[0]
    nnz = hyperedge_index.shape[1]

    np_ = _round_up(max(num_nodes, num_edges), BLK)
    nnzp = _round_up(nnz, 2 * KI)
    pad_slot = max(num_nodes, num_edges) + 16

    x_p = jnp.zeros((np_, D), jnp.float32).at[:num_nodes].set(x)
    y_p = jnp.zeros((np_, D), jnp.float32).at[:num_edges].set(y)
    xt_in = x_p.T
    yt_in = y_p.T
    w1t = W1.T
    w2t = W2.T
    padv = jnp.full((nnzp - nnz,), pad_slot, jnp.int32)
    src = jnp.concatenate([hyperedge_index[0].astype(jnp.int32), padv])
    dst = jnp.concatenate([hyperedge_index[1].astype(jnp.int32), padv])

    grid = np_ // BLK
    tcol_spec = pl.BlockSpec((D, BLK), lambda i: (0, i))
    nrow_spec = pl.BlockSpec((BLK, D), lambda i: (i, 0))
    deg_spec = pl.BlockSpec((1, 1, BLK), lambda i: (i, 0, 0))
    w_spec = pl.BlockSpec((D, D), lambda i: (0, 0))

    xwt = pl.pallas_call(
        _stage_a,
        grid=(grid,),
        in_specs=[tcol_spec, w_spec],
        out_specs=tcol_spec,
        out_shape=jax.ShapeDtypeStruct((D, np_), jnp.float32),
    )(xt_in, w1t)

    eacct, dege, degv = _sc_pass(np_, nnzp, True)(xwt, src, dst)

    etop, ewt, ewselft = pl.pallas_call(
        _stage_b,
        grid=(grid,),
        in_specs=[tcol_spec, deg_spec, tcol_spec, tcol_spec, tcol_spec,
                  w_spec],
        out_specs=[nrow_spec, tcol_spec, tcol_spec],
        out_shape=[jax.ShapeDtypeStruct((np_, D), jnp.float32),
                   jax.ShapeDtypeStruct((D, np_), jnp.float32),
                   jax.ShapeDtypeStruct((D, np_), jnp.float32)],
    )(eacct, dege.reshape(grid, 1, BLK), yt_in, xwt, xt_in, w2t)

    nacct = _sc_pass(np_, nnzp, False)(ewt, dst, src)

    n = pl.pallas_call(
        _stage_c,
        grid=(grid,),
        in_specs=[tcol_spec, tcol_spec, deg_spec],
        out_specs=nrow_spec,
        out_shape=jax.ShapeDtypeStruct((np_, D), jnp.float32),
    )(nacct, ewselft, degv.reshape(grid, 1, BLK))

    return (n[:num_nodes], etop[:num_edges])


# speedup vs baseline: 1.0051x; 1.0051x over previous
import functools

import jax
import jax.numpy as jnp
from jax import lax
from jax.experimental import pallas as pl
from jax.experimental.pallas import tpu as pltpu
from jax.experimental.pallas import tpu_sc as plsc

NC, NS = 2, 16
NW = NC * NS
K = 128
D = 128
BLK = 2048


def _round_up(a, b):
    return (a + b - 1) // b * b


def _mm_a(x_ref, w_ref, o_ref):
    o_ref[...] = jnp.dot(x_ref[...], w_ref[...],
                         preferred_element_type=jnp.float32)


def _stage_b(e0, e1, d0, d1, yb, xwb, xb, w2, etop_o, ew_o, ewself_o):
    deg = jnp.maximum(d0[...] + d1[...], 1.0)
    eagg = (e0[...] + e1[...]) / deg
    etop = jnp.maximum(eagg + yb[...], 0.0)
    etop_o[...] = etop
    ew_o[...] = jnp.dot(etop, w2[...], preferred_element_type=jnp.float32)
    eself = jnp.maximum(xwb[...] + xb[...], 0.0)
    ewself_o[...] = jnp.dot(eself, w2[...], preferred_element_type=jnp.float32)


def _stage_c(n0, n1, ews, v0, v1, n_o):
    deg = v0[...] + v1[...] + 1.0
    acc = n0[...] + n1[...] + ews[...]
    n_o[...] = jnp.maximum(acc / deg, 0.0)


def _sc_pass1(np_, chunks):
    z = np_ // NS

    def body(xw_hbm, sd_hbm, z2d, z1d,
             eacc_out, dege_out, degv_out,
             ib0, ib1, rows, ones_v, eacc_sh, dege_sh, degv_sh, semr, semd):
        c = lax.axis_index("c")
        s = lax.axis_index("s")
        zoff = s * z
        pltpu.sync_copy(z2d.at[pl.ds(zoff, z)], eacc_sh.at[pl.ds(zoff, z)])
        pltpu.sync_copy(z1d.at[pl.ds(zoff, z)], dege_sh.at[pl.ds(zoff, z)])
        pltpu.sync_copy(z1d.at[pl.ds(zoff, z)], degv_sh.at[pl.ds(zoff, z)])
        for i in range(K // 16):
            ones_v[pl.ds(i * 16, 16)] = jnp.ones((16,), jnp.float32)
        plsc.subcore_barrier()
        wid = s * NC + c
        base = wid * (chunks * K)

        def pair(p, carry):
            @pl.when(p >= 1)
            def _():
                for ib in (ib0, ib1):
                    pltpu.make_async_copy(
                        ones_v, dege_sh.at[ib.at[1]], semd).wait()
                    pltpu.make_async_copy(
                        ones_v, degv_sh.at[ib.at[0]], semd).wait()
            for phase, ib in ((0, ib0), (1, ib1)):
                off = pl.multiple_of(base + (2 * p + phase) * K, 8)
                pltpu.sync_copy(sd_hbm.at[:, pl.ds(off, K)], ib)
                pltpu.async_copy(xw_hbm.at[ib.at[0]], rows, semr).wait()
                pltpu.sync_copy(rows, eacc_sh.at[ib.at[1]], add=True)
                pltpu.async_copy(ones_v, dege_sh.at[ib.at[1]], semd, add=True)
                pltpu.async_copy(ones_v, degv_sh.at[ib.at[0]], semd, add=True)
            return carry

        lax.fori_loop(0, chunks // 2, pair, 0)
        for ib in (ib0, ib1):
            pltpu.make_async_copy(ones_v, dege_sh.at[ib.at[1]], semd).wait()
            pltpu.make_async_copy(ones_v, degv_sh.at[ib.at[0]], semd).wait()
        plsc.subcore_barrier()
        pltpu.sync_copy(eacc_sh.at[pl.ds(zoff, z)],
                        eacc_out.at[c, pl.ds(zoff, z)])
        pltpu.sync_copy(dege_sh.at[pl.ds(zoff, z)],
                        dege_out.at[c, pl.ds(zoff, z)])
        pltpu.sync_copy(degv_sh.at[pl.ds(zoff, z)],
                        degv_out.at[c, pl.ds(zoff, z)])

    return pl.kernel(
        body,
        out_type=(
            jax.ShapeDtypeStruct((NC, np_, D), jnp.float32),
            jax.ShapeDtypeStruct((NC, np_), jnp.float32),
            jax.ShapeDtypeStruct((NC, np_), jnp.float32),
        ),
        mesh=plsc.VectorSubcoreMesh(core_axis_name="c", subcore_axis_name="s"),
        scratch_types=[
            pltpu.VMEM((2, K), jnp.int32),
            pltpu.VMEM((2, K), jnp.int32),
            pltpu.VMEM((K, D), jnp.float32),
            pltpu.VMEM((K,), jnp.float32),
            pltpu.VMEM_SHARED((np_, D), jnp.float32),
            pltpu.VMEM_SHARED((np_,), jnp.float32),
            pltpu.VMEM_SHARED((np_,), jnp.float32),
            pltpu.SemaphoreType.DMA,
            pltpu.SemaphoreType.DMA,
        ],
    )


def _sc_pass2(np_, chunks):
    z = np_ // NS

    def body(ew_hbm, sd_hbm, z2d,
             nacc_out,
             ib0, rows, nacc_sh, semr):
        c = lax.axis_index("c")
        s = lax.axis_index("s")
        zoff = s * z
        pltpu.sync_copy(z2d.at[pl.ds(zoff, z)], nacc_sh.at[pl.ds(zoff, z)])
        plsc.subcore_barrier()
        wid = s * NC + c
        base = wid * (chunks * K)

        def step(j, carry):
            off = pl.multiple_of(base + j * K, 8)
            pltpu.sync_copy(sd_hbm.at[:, pl.ds(off, K)], ib0)
            pltpu.async_copy(ew_hbm.at[ib0.at[1]], rows, semr).wait()
            pltpu.sync_copy(rows, nacc_sh.at[ib0.at[0]], add=True)
            return carry

        lax.fori_loop(0, chunks, step, 0)
        plsc.subcore_barrier()
        pltpu.sync_copy(nacc_sh.at[pl.ds(zoff, z)],
                        nacc_out.at[c, pl.ds(zoff, z)])

    return pl.kernel(
        body,
        out_type=jax.ShapeDtypeStruct((NC, np_, D), jnp.float32),
        mesh=plsc.VectorSubcoreMesh(core_axis_name="c", subcore_axis_name="s"),
        scratch_types=[
            pltpu.VMEM((2, K), jnp.int32),
            pltpu.VMEM((K, D), jnp.float32),
            pltpu.VMEM_SHARED((np_, D), jnp.float32),
            pltpu.SemaphoreType.DMA,
        ],
    )


def kernel(x, y, hyperedge_index, W1, W2):
    num_nodes = x.shape[0]
    num_edges = y.shape[0]
    nnz = hyperedge_index.shape[1]

    np_ = _round_up(max(num_nodes, num_edges), BLK)
    nnzp = _round_up(nnz, 2 * K * NW)
    chunks = nnzp // (K * NW)
    pad_slot = max(num_nodes, num_edges) + 16

    x_p = jnp.zeros((np_, D), jnp.float32).at[:num_nodes].set(x)
    y_p = jnp.zeros((np_, D), jnp.float32).at[:num_edges].set(y)
    padv = jnp.full((2, nnzp - nnz), pad_slot, jnp.int32)
    sd = jnp.concatenate([hyperedge_index.astype(jnp.int32), padv], axis=1)
    z2d = jnp.zeros((np_, D), jnp.float32)
    z1d = jnp.zeros((np_,), jnp.float32)

    grid = np_ // BLK
    row_spec = pl.BlockSpec((BLK, D), lambda i: (i, 0))
    col_spec = pl.BlockSpec((BLK, 1), lambda i: (i, 0))
    w_spec = pl.BlockSpec((D, D), lambda i: (0, 0))

    xw_p = pl.pallas_call(
        _mm_a,
        grid=(grid,),
        in_specs=[row_spec, w_spec],
        out_specs=row_spec,
        out_shape=jax.ShapeDtypeStruct((np_, D), jnp.float32),
    )(x_p, W1)

    eacc, dege, degv = _sc_pass1(np_, chunks)(xw_p, sd, z2d, z1d)

    etop, ew, ewself = pl.pallas_call(
        _stage_b,
        grid=(grid,),
        in_specs=[row_spec, row_spec, col_spec, col_spec,
                  row_spec, row_spec, row_spec, w_spec],
        out_specs=[row_spec, row_spec, row_spec],
        out_shape=[jax.ShapeDtypeStruct((np_, D), jnp.float32)] * 3,
    )(eacc[0], eacc[1], dege[0].reshape(np_, 1), dege[1].reshape(np_, 1),
      y_p, xw_p, x_p, W2)

    nacc = _sc_pass2(np_, chunks)(ew, sd, z2d)

    n = pl.pallas_call(
        _stage_c,
        grid=(grid,),
        in_specs=[row_spec, row_spec, row_spec, col_spec, col_spec],
        out_specs=row_spec,
        out_shape=jax.ShapeDtypeStruct((np_, D), jnp.float32),
    )(nacc[0], nacc[1], ewself,
      degv[0].reshape(np_, 1), degv[1].reshape(np_, 1))

    return (n[:num_nodes], etop[:num_edges])


# speedup vs baseline: 1.2694x; 1.2629x over previous
import functools

import jax
import jax.numpy as jnp
from jax import lax
from jax.experimental import pallas as pl
from jax.experimental.pallas import tpu as pltpu
from jax.experimental.pallas import tpu_sc as plsc

NC, NS = 2, 16
NW = NC * NS
K = 128
D = 128
BLK = 2048


def _round_up(a, b):
    return (a + b - 1) // b * b


def _mm_a(x_ref, w_ref, o_ref):
    o_ref[...] = jnp.dot(x_ref[...], w_ref[...],
                         preferred_element_type=jnp.float32)


def _stage_b(e0, e1, d0, d1, yb, xwb, xb, w2, etop_o, ew_o, ewself_o):
    deg = jnp.maximum(d0[...] + d1[...], 1.0)
    eagg = (e0[...] + e1[...]) / deg
    etop = jnp.maximum(eagg + yb[...], 0.0)
    etop_o[...] = etop
    ew_o[...] = jnp.dot(etop, w2[...], preferred_element_type=jnp.float32)
    eself = jnp.maximum(xwb[...] + xb[...], 0.0)
    ewself_o[...] = jnp.dot(eself, w2[...], preferred_element_type=jnp.float32)


def _stage_c(n0, n1, ews, v0, v1, n_o):
    deg = v0[...] + v1[...] + 1.0
    acc = n0[...] + n1[...] + ews[...]
    n_o[...] = jnp.maximum(acc / deg, 0.0)


def _sc_pass1(np_, chunks):
    z = np_ // NS

    def body(xw_hbm, src_hbm, dst_hbm, z2d, z1d,
             eacc_out, dege_out, degv_out,
             idx_s, idx_d, rows, ones_v, eacc_sh, dege_sh, degv_sh, sem):
        c = lax.axis_index("c")
        s = lax.axis_index("s")
        zoff = s * z
        pltpu.sync_copy(z2d.at[pl.ds(zoff, z)], eacc_sh.at[pl.ds(zoff, z)])
        pltpu.sync_copy(z1d.at[pl.ds(zoff, z)], dege_sh.at[pl.ds(zoff, z)])
        pltpu.sync_copy(z1d.at[pl.ds(zoff, z)], degv_sh.at[pl.ds(zoff, z)])
        for i in range(K // 16):
            ones_v[pl.ds(i * 16, 16)] = jnp.ones((16,), jnp.float32)
        plsc.subcore_barrier()
        wid = s * NC + c
        base = wid * (chunks * K)

        def step(j, carry):
            off = pl.multiple_of(base + j * K, 8)
            pltpu.sync_copy(src_hbm.at[pl.ds(off, K)], idx_s)
            pltpu.sync_copy(dst_hbm.at[pl.ds(off, K)], idx_d)
            pltpu.async_copy(xw_hbm.at[idx_s], rows, sem).wait()
            pltpu.sync_copy(rows, eacc_sh.at[idx_d], add=True)
            pltpu.sync_copy(ones_v, dege_sh.at[idx_d], add=True)
            pltpu.sync_copy(ones_v, degv_sh.at[idx_s], add=True)
            return carry

        lax.fori_loop(0, chunks, step, 0)
        plsc.subcore_barrier()
        pltpu.sync_copy(eacc_sh.at[pl.ds(zoff, z)],
                        eacc_out.at[c, pl.ds(zoff, z)])
        pltpu.sync_copy(dege_sh.at[pl.ds(zoff, z)],
                        dege_out.at[c, pl.ds(zoff, z)])
        pltpu.sync_copy(degv_sh.at[pl.ds(zoff, z)],
                        degv_out.at[c, pl.ds(zoff, z)])

    return pl.kernel(
        body,
        out_type=(
            jax.ShapeDtypeStruct((NC, np_, D), jnp.float32),
            jax.ShapeDtypeStruct((NC, np_), jnp.float32),
            jax.ShapeDtypeStruct((NC, np_), jnp.float32),
        ),
        mesh=plsc.VectorSubcoreMesh(core_axis_name="c", subcore_axis_name="s"),
        scratch_types=[
            pltpu.VMEM((K,), jnp.int32),
            pltpu.VMEM((K,), jnp.int32),
            pltpu.VMEM((K, D), jnp.float32),
            pltpu.VMEM((K,), jnp.float32),
            pltpu.VMEM_SHARED((np_, D), jnp.float32),
            pltpu.VMEM_SHARED((np_,), jnp.float32),
            pltpu.VMEM_SHARED((np_,), jnp.float32),
            pltpu.SemaphoreType.DMA,
        ],
    )


def _sc_pass2(np_, chunks):
    z = np_ // NS

    def body(ew_hbm, src_hbm, dst_hbm, z2d,
             nacc_out,
             idx_s, idx_d, rows, nacc_sh, sem):
        c = lax.axis_index("c")
        s = lax.axis_index("s")
        zoff = s * z
        pltpu.sync_copy(z2d.at[pl.ds(zoff, z)], nacc_sh.at[pl.ds(zoff, z)])
        plsc.subcore_barrier()
        wid = s * NC + c
        base = wid * (chunks * K)

        def step(j, carry):
            off = pl.multiple_of(base + j * K, 8)
            pltpu.sync_copy(src_hbm.at[pl.ds(off, K)], idx_s)
            pltpu.sync_copy(dst_hbm.at[pl.ds(off, K)], idx_d)
            pltpu.async_copy(ew_hbm.at[idx_d], rows, sem).wait()
            pltpu.sync_copy(rows, nacc_sh.at[idx_s], add=True)
            return carry

        lax.fori_loop(0, chunks, step, 0)
        plsc.subcore_barrier()
        pltpu.sync_copy(nacc_sh.at[pl.ds(zoff, z)],
                        nacc_out.at[c, pl.ds(zoff, z)])

    return pl.kernel(
        body,
        out_type=jax.ShapeDtypeStruct((NC, np_, D), jnp.float32),
        mesh=plsc.VectorSubcoreMesh(core_axis_name="c", subcore_axis_name="s"),
        scratch_types=[
            pltpu.VMEM((K,), jnp.int32),
            pltpu.VMEM((K,), jnp.int32),
            pltpu.VMEM((K, D), jnp.float32),
            pltpu.VMEM_SHARED((np_, D), jnp.float32),
            pltpu.SemaphoreType.DMA,
        ],
    )


def kernel(x, y, hyperedge_index, W1, W2):
    num_nodes = x.shape[0]
    num_edges = y.shape[0]
    nnz = hyperedge_index.shape[1]

    np_ = _round_up(max(num_nodes, num_edges), BLK)
    nnzp = _round_up(nnz, K * NW)
    chunks = nnzp // (K * NW)
    pad_slot = max(num_nodes, num_edges) + 16

    x_p = jnp.zeros((np_, D), jnp.float32).at[:num_nodes].set(x)
    y_p = jnp.zeros((np_, D), jnp.float32).at[:num_edges].set(y)
    padv = jnp.full((nnzp - nnz,), pad_slot, jnp.int32)
    src = jnp.concatenate([hyperedge_index[0].astype(jnp.int32), padv])
    dst = jnp.concatenate([hyperedge_index[1].astype(jnp.int32), padv])
    z2d = jnp.zeros((np_, D), jnp.float32)
    z1d = jnp.zeros((np_,), jnp.float32)

    grid = np_ // BLK
    row_spec = pl.BlockSpec((BLK, D), lambda i: (i, 0))
    col_spec = pl.BlockSpec((BLK, 1), lambda i: (i, 0))
    w_spec = pl.BlockSpec((D, D), lambda i: (0, 0))

    xw_p = pl.pallas_call(
        _mm_a,
        grid=(grid,),
        in_specs=[row_spec, w_spec],
        out_specs=row_spec,
        out_shape=jax.ShapeDtypeStruct((np_, D), jnp.float32),
    )(x_p, W1)

    eacc, dege, degv = _sc_pass1(np_, chunks)(xw_p, src, dst, z2d, z1d)

    etop, ew, ewself = pl.pallas_call(
        _stage_b,
        grid=(grid,),
        in_specs=[row_spec, row_spec, col_spec, col_spec,
                  row_spec, row_spec, row_spec, w_spec],
        out_specs=[row_spec, row_spec, row_spec],
        out_shape=[jax.ShapeDtypeStruct((np_, D), jnp.float32)] * 3,
    )(eacc[0], eacc[1], dege[0].reshape(np_, 1), dege[1].reshape(np_, 1),
      y_p, xw_p, x_p, W2)

    nacc = _sc_pass2(np_, chunks)(ew, src, dst, z2d)

    n = pl.pallas_call(
        _stage_c,
        grid=(grid,),
        in_specs=[row_spec, row_spec, row_spec, col_spec, col_spec],
        out_specs=row_spec,
        out_shape=jax.ShapeDtypeStruct((np_, D), jnp.float32),
    )(nacc[0], nacc[1], ewself,
      degv[0].reshape(np_, 1), degv[1].reshape(np_, 1))

    return (n[:num_nodes], etop[:num_edges])


# speedup vs baseline: 1.4537x; 1.1452x over previous
import functools

import jax
import jax.numpy as jnp
from jax import lax
from jax.experimental import pallas as pl
from jax.experimental.pallas import tpu as pltpu
from jax.experimental.pallas import tpu_sc as plsc

NC, NS = 2, 16
NW = NC * NS
K = 128
D = 128
BLK = 2048


def _round_up(a, b):
    return (a + b - 1) // b * b


def _mm_a(x_ref, w_ref, o_ref):
    o_ref[...] = jnp.dot(x_ref[...], w_ref[...],
                         preferred_element_type=jnp.float32)


def _stage_b(e0, e1, d0, d1, yb, xwb, xb, w2, etop_o, ew_o, ewself_o):
    deg = jnp.maximum(d0[...] + d1[...], 1.0)
    eagg = (e0[...] + e1[...]) / deg
    etop = jnp.maximum(eagg + yb[...], 0.0)
    etop_o[...] = etop
    ew_o[...] = jnp.dot(etop, w2[...], preferred_element_type=jnp.float32)
    eself = jnp.maximum(xwb[...] + xb[...], 0.0)
    ewself_o[...] = jnp.dot(eself, w2[...], preferred_element_type=jnp.float32)


def _stage_c(n0, n1, ews, v0, v1, n_o):
    deg = v0[...] + v1[...] + 1.0
    acc = n0[...] + n1[...] + ews[...]
    n_o[...] = jnp.maximum(acc / deg, 0.0)


def _sc_pass1(np_, chunks):
    z = np_ // NS

    def body(xw_hbm, src_hbm, dst_hbm, z2d, z1d,
             eacc_out, dege_out, degv_out,
             idx_s, idx_d, rows, ones_v, eacc_sh, dege_sh, degv_sh, sem):
        c = lax.axis_index("c")
        s = lax.axis_index("s")
        zoff = s * z
        pltpu.sync_copy(z2d.at[pl.ds(zoff, z)], eacc_sh.at[pl.ds(zoff, z)])
        pltpu.sync_copy(z1d.at[pl.ds(zoff, z)], dege_sh.at[pl.ds(zoff, z)])
        pltpu.sync_copy(z1d.at[pl.ds(zoff, z)], degv_sh.at[pl.ds(zoff, z)])
        for i in range(K // 16):
            ones_v[pl.ds(i * 16, 16)] = jnp.ones((16,), jnp.float32)
        plsc.subcore_barrier()
        wid = s * NC + c
        pltpu.sync_copy(src_hbm.at[pl.ds(wid * chunks, chunks)], idx_s)
        pltpu.sync_copy(dst_hbm.at[pl.ds(wid * chunks, chunks)], idx_d)

        def step(j, carry):
            pltpu.async_copy(xw_hbm.at[idx_s.at[j]], rows, sem).wait()
            pltpu.sync_copy(rows, eacc_sh.at[idx_d.at[j]], add=True)
            pltpu.sync_copy(ones_v, dege_sh.at[idx_d.at[j]], add=True)
            pltpu.sync_copy(ones_v, degv_sh.at[idx_s.at[j]], add=True)
            return carry

        lax.fori_loop(0, chunks, step, 0)
        plsc.subcore_barrier()
        pltpu.sync_copy(eacc_sh.at[pl.ds(zoff, z)],
                        eacc_out.at[c, pl.ds(zoff, z)])
        pltpu.sync_copy(dege_sh.at[pl.ds(zoff, z)],
                        dege_out.at[c, pl.ds(zoff, z)])
        pltpu.sync_copy(degv_sh.at[pl.ds(zoff, z)],
                        degv_out.at[c, pl.ds(zoff, z)])

    return pl.kernel(
        body,
        out_type=(
            jax.ShapeDtypeStruct((NC, np_, D), jnp.float32),
            jax.ShapeDtypeStruct((NC, np_), jnp.float32),
            jax.ShapeDtypeStruct((NC, np_), jnp.float32),
        ),
        mesh=plsc.VectorSubcoreMesh(core_axis_name="c", subcore_axis_name="s"),
        scratch_types=[
            pltpu.VMEM((chunks, K), jnp.int32),
            pltpu.VMEM((chunks, K), jnp.int32),
            pltpu.VMEM((K, D), jnp.float32),
            pltpu.VMEM((K,), jnp.float32),
            pltpu.VMEM_SHARED((np_, D), jnp.float32),
            pltpu.VMEM_SHARED((np_,), jnp.float32),
            pltpu.VMEM_SHARED((np_,), jnp.float32),
            pltpu.SemaphoreType.DMA,
        ],
        compiler_params=pltpu.CompilerParams(use_tc_tiling_on_sc=False),
    )


def _sc_pass2(np_, chunks):
    z = np_ // NS

    def body(ew_hbm, src_hbm, dst_hbm, z2d,
             nacc_out,
             idx_s, idx_d, rows, nacc_sh, sem):
        c = lax.axis_index("c")
        s = lax.axis_index("s")
        zoff = s * z
        pltpu.sync_copy(z2d.at[pl.ds(zoff, z)], nacc_sh.at[pl.ds(zoff, z)])
        plsc.subcore_barrier()
        wid = s * NC + c
        pltpu.sync_copy(src_hbm.at[pl.ds(wid * chunks, chunks)], idx_s)
        pltpu.sync_copy(dst_hbm.at[pl.ds(wid * chunks, chunks)], idx_d)

        def step(j, carry):
            pltpu.async_copy(ew_hbm.at[idx_d.at[j]], rows, sem).wait()
            pltpu.sync_copy(rows, nacc_sh.at[idx_s.at[j]], add=True)
            return carry

        lax.fori_loop(0, chunks, step, 0)
        plsc.subcore_barrier()
        pltpu.sync_copy(nacc_sh.at[pl.ds(zoff, z)],
                        nacc_out.at[c, pl.ds(zoff, z)])

    return pl.kernel(
        body,
        out_type=jax.ShapeDtypeStruct((NC, np_, D), jnp.float32),
        mesh=plsc.VectorSubcoreMesh(core_axis_name="c", subcore_axis_name="s"),
        scratch_types=[
            pltpu.VMEM((chunks, K), jnp.int32),
            pltpu.VMEM((chunks, K), jnp.int32),
            pltpu.VMEM((K, D), jnp.float32),
            pltpu.VMEM_SHARED((np_, D), jnp.float32),
            pltpu.SemaphoreType.DMA,
        ],
        compiler_params=pltpu.CompilerParams(use_tc_tiling_on_sc=False),
    )


def kernel(x, y, hyperedge_index, W1, W2):
    num_nodes = x.shape[0]
    num_edges = y.shape[0]
    nnz = hyperedge_index.shape[1]

    np_ = _round_up(max(num_nodes, num_edges), BLK)
    nnzp = _round_up(nnz, K * NW)
    chunks = nnzp // (K * NW)
    pad_slot = max(num_nodes, num_edges) + 16

    x_p = jnp.zeros((np_, D), jnp.float32).at[:num_nodes].set(x)
    y_p = jnp.zeros((np_, D), jnp.float32).at[:num_edges].set(y)
    padv = jnp.full((nnzp - nnz,), pad_slot, jnp.int32)
    src = jnp.concatenate(
        [hyperedge_index[0].astype(jnp.int32), padv]).reshape(nnzp // K, K)
    dst = jnp.concatenate(
        [hyperedge_index[1].astype(jnp.int32), padv]).reshape(nnzp // K, K)
    z2d = jnp.zeros((np_, D), jnp.float32)
    z1d = jnp.zeros((np_,), jnp.float32)

    grid = np_ // BLK
    row_spec = pl.BlockSpec((BLK, D), lambda i: (i, 0))
    col_spec = pl.BlockSpec((BLK, 1), lambda i: (i, 0))
    w_spec = pl.BlockSpec((D, D), lambda i: (0, 0))

    xw_p = pl.pallas_call(
        _mm_a,
        grid=(grid,),
        in_specs=[row_spec, w_spec],
        out_specs=row_spec,
        out_shape=jax.ShapeDtypeStruct((np_, D), jnp.float32),
    )(x_p, W1)

    eacc, dege, degv = _sc_pass1(np_, chunks)(xw_p, src, dst, z2d, z1d)

    etop, ew, ewself = pl.pallas_call(
        _stage_b,
        grid=(grid,),
        in_specs=[row_spec, row_spec, col_spec, col_spec,
                  row_spec, row_spec, row_spec, w_spec],
        out_specs=[row_spec, row_spec, row_spec],
        out_shape=[jax.ShapeDtypeStruct((np_, D), jnp.float32)] * 3,
    )(eacc[0], eacc[1], dege[0].reshape(np_, 1), dege[1].reshape(np_, 1),
      y_p, xw_p, x_p, W2)

    nacc = _sc_pass2(np_, chunks)(ew, src, dst, z2d)

    n = pl.pallas_call(
        _stage_c,
        grid=(grid,),
        in_specs=[row_spec, row_spec, row_spec, col_spec, col_spec],
        out_specs=row_spec,
        out_shape=jax.ShapeDtypeStruct((np_, D), jnp.float32),
    )(nacc[0], nacc[1], ewself,
      degv[0].reshape(np_, 1), degv[1].reshape(np_, 1))

    return (n[:num_nodes], etop[:num_edges])


# speedup vs baseline: 1.4814x; 1.0190x over previous
import functools

import jax
import jax.numpy as jnp
from jax import lax
from jax.experimental import pallas as pl
from jax.experimental.pallas import tpu as pltpu
from jax.experimental.pallas import tpu_sc as plsc

NC, NS = 2, 16
NW = NC * NS
K = 128
D = 128
BLK = 2048


def _round_up(a, b):
    return (a + b - 1) // b * b


def _mm_a(x_ref, w_ref, o_ref):
    o_ref[...] = jnp.dot(x_ref[...], w_ref[...],
                         preferred_element_type=jnp.float32)


def _stage_b(e0, e1, d0, d1, yb, xwb, xb, w2, etop_o, ew_o, ewself_o):
    deg = jnp.maximum(d0[...] + d1[...], 1.0)
    eagg = (e0[...] + e1[...]) / deg
    etop = jnp.maximum(eagg + yb[...], 0.0)
    etop_o[...] = etop
    ew_o[...] = jnp.dot(etop, w2[...], preferred_element_type=jnp.float32)
    eself = jnp.maximum(xwb[...] + xb[...], 0.0)
    ewself_o[...] = jnp.dot(eself, w2[...], preferred_element_type=jnp.float32)


def _stage_c(n0, n1, ews, v0, v1, n_o):
    deg = v0[...] + v1[...] + 1.0
    acc = n0[...] + n1[...] + ews[...]
    n_o[...] = jnp.maximum(acc / deg, 0.0)


def _sc_pass1(np_, chunks):
    z = np_ // NS

    def body(xw_hbm, src_hbm, dst_hbm, z2d, z1d,
             eacc_out, dege_out, degv_out,
             idx_s, idx_d, rows, ones_v, eacc_sh, dege_sh, degv_sh,
             sem, semd):
        c = lax.axis_index("c")
        s = lax.axis_index("s")
        zoff = s * z
        pltpu.sync_copy(z2d.at[pl.ds(zoff, z)], eacc_sh.at[pl.ds(zoff, z)])
        pltpu.sync_copy(z1d.at[pl.ds(zoff, z)], dege_sh.at[pl.ds(zoff, z)])
        pltpu.sync_copy(z1d.at[pl.ds(zoff, z)], degv_sh.at[pl.ds(zoff, z)])
        for i in range(K // 16):
            ones_v[pl.ds(i * 16, 16)] = jnp.ones((16,), jnp.float32)
        plsc.subcore_barrier()
        wid = s * NC + c
        pltpu.sync_copy(src_hbm.at[pl.ds(wid * chunks, chunks)], idx_s)
        pltpu.sync_copy(dst_hbm.at[pl.ds(wid * chunks, chunks)], idx_d)

        def step(j, carry):
            pltpu.async_copy(xw_hbm.at[idx_s.at[j]], rows, sem).wait()
            pltpu.sync_copy(rows, eacc_sh.at[idx_d.at[j]], add=True)
            pltpu.async_copy(ones_v, dege_sh.at[idx_d.at[j]], semd, add=True)
            pltpu.async_copy(ones_v, degv_sh.at[idx_s.at[j]], semd, add=True)
            return carry

        lax.fori_loop(0, chunks, step, 0)

        def drain(j, carry):
            pltpu.make_async_copy(ones_v, dege_sh.at[idx_d.at[j]], semd).wait()
            pltpu.make_async_copy(ones_v, degv_sh.at[idx_s.at[j]], semd).wait()
            return carry

        lax.fori_loop(0, chunks, drain, 0)
        plsc.subcore_barrier()
        pltpu.sync_copy(eacc_sh.at[pl.ds(zoff, z)],
                        eacc_out.at[c, pl.ds(zoff, z)])
        pltpu.sync_copy(dege_sh.at[pl.ds(zoff, z)],
                        dege_out.at[c, pl.ds(zoff, z)])
        pltpu.sync_copy(degv_sh.at[pl.ds(zoff, z)],
                        degv_out.at[c, pl.ds(zoff, z)])

    return pl.kernel(
        body,
        out_type=(
            jax.ShapeDtypeStruct((NC, np_, D), jnp.float32),
            jax.ShapeDtypeStruct((NC, np_), jnp.float32),
            jax.ShapeDtypeStruct((NC, np_), jnp.float32),
        ),
        mesh=plsc.VectorSubcoreMesh(core_axis_name="c", subcore_axis_name="s"),
        scratch_types=[
            pltpu.VMEM((chunks, K), jnp.int32),
            pltpu.VMEM((chunks, K), jnp.int32),
            pltpu.VMEM((K, D), jnp.float32),
            pltpu.VMEM((K,), jnp.float32),
            pltpu.VMEM_SHARED((np_, D), jnp.float32),
            pltpu.VMEM_SHARED((np_,), jnp.float32),
            pltpu.VMEM_SHARED((np_,), jnp.float32),
            pltpu.SemaphoreType.DMA,
            pltpu.SemaphoreType.DMA,
        ],
        compiler_params=pltpu.CompilerParams(use_tc_tiling_on_sc=False),
    )


def _sc_pass2(np_, chunks):
    z = np_ // NS

    def body(ew_hbm, src_hbm, dst_hbm, z2d,
             nacc_out,
             idx_s, idx_d, rows, nacc_sh, sem):
        c = lax.axis_index("c")
        s = lax.axis_index("s")
        zoff = s * z
        pltpu.sync_copy(z2d.at[pl.ds(zoff, z)], nacc_sh.at[pl.ds(zoff, z)])
        plsc.subcore_barrier()
        wid = s * NC + c
        pltpu.sync_copy(src_hbm.at[pl.ds(wid * chunks, chunks)], idx_s)
        pltpu.sync_copy(dst_hbm.at[pl.ds(wid * chunks, chunks)], idx_d)

        def step(j, carry):
            pltpu.async_copy(ew_hbm.at[idx_d.at[j]], rows, sem).wait()
            pltpu.sync_copy(rows, nacc_sh.at[idx_s.at[j]], add=True)
            return carry

        lax.fori_loop(0, chunks, step, 0)
        plsc.subcore_barrier()
        pltpu.sync_copy(nacc_sh.at[pl.ds(zoff, z)],
                        nacc_out.at[c, pl.ds(zoff, z)])

    return pl.kernel(
        body,
        out_type=jax.ShapeDtypeStruct((NC, np_, D), jnp.float32),
        mesh=plsc.VectorSubcoreMesh(core_axis_name="c", subcore_axis_name="s"),
        scratch_types=[
            pltpu.VMEM((chunks, K), jnp.int32),
            pltpu.VMEM((chunks, K), jnp.int32),
            pltpu.VMEM((K, D), jnp.float32),
            pltpu.VMEM_SHARED((np_, D), jnp.float32),
            pltpu.SemaphoreType.DMA,
        ],
        compiler_params=pltpu.CompilerParams(use_tc_tiling_on_sc=False),
    )


def kernel(x, y, hyperedge_index, W1, W2):
    num_nodes = x.shape[0]
    num_edges = y.shape[0]
    nnz = hyperedge_index.shape[1]

    np_ = _round_up(max(num_nodes, num_edges), BLK)
    nnzp = _round_up(nnz, K * NW)
    chunks = nnzp // (K * NW)
    pad_slot = max(num_nodes, num_edges) + 16

    x_p = jnp.zeros((np_, D), jnp.float32).at[:num_nodes].set(x)
    y_p = jnp.zeros((np_, D), jnp.float32).at[:num_edges].set(y)
    padv = jnp.full((nnzp - nnz,), pad_slot, jnp.int32)
    src = jnp.concatenate(
        [hyperedge_index[0].astype(jnp.int32), padv]).reshape(nnzp // K, K)
    dst = jnp.concatenate(
        [hyperedge_index[1].astype(jnp.int32), padv]).reshape(nnzp // K, K)
    z2d = jnp.zeros((np_, D), jnp.float32)
    z1d = jnp.zeros((np_,), jnp.float32)

    grid = np_ // BLK
    row_spec = pl.BlockSpec((BLK, D), lambda i: (i, 0))
    col_spec = pl.BlockSpec((BLK, 1), lambda i: (i, 0))
    w_spec = pl.BlockSpec((D, D), lambda i: (0, 0))

    xw_p = pl.pallas_call(
        _mm_a,
        grid=(grid,),
        in_specs=[row_spec, w_spec],
        out_specs=row_spec,
        out_shape=jax.ShapeDtypeStruct((np_, D), jnp.float32),
    )(x_p, W1)

    eacc, dege, degv = _sc_pass1(np_, chunks)(xw_p, src, dst, z2d, z1d)

    etop, ew, ewself = pl.pallas_call(
        _stage_b,
        grid=(grid,),
        in_specs=[row_spec, row_spec, col_spec, col_spec,
                  row_spec, row_spec, row_spec, w_spec],
        out_specs=[row_spec, row_spec, row_spec],
        out_shape=[jax.ShapeDtypeStruct((np_, D), jnp.float32)] * 3,
    )(eacc[0], eacc[1], dege[0].reshape(np_, 1), dege[1].reshape(np_, 1),
      y_p, xw_p, x_p, W2)

    nacc = _sc_pass2(np_, chunks)(ew, src, dst, z2d)

    n = pl.pallas_call(
        _stage_c,
        grid=(grid,),
        in_specs=[row_spec, row_spec, row_spec, col_spec, col_spec],
        out_specs=row_spec,
        out_shape=jax.ShapeDtypeStruct((np_, D), jnp.float32),
    )(nacc[0], nacc[1], ewself,
      degv[0].reshape(np_, 1), degv[1].reshape(np_, 1))

    return (n[:num_nodes], etop[:num_edges])
